# Initial kernel scaffold; baseline (speedup 1.0000x reference)
import jax
import jax.numpy as jnp
from jax.experimental import pallas as pl


def kernel(feats, u, v, in_W1, in_g1, in_b1, in_W2, in_g2, in_b2, in_Wt, in_gt, in_bt, fc1_W, fc1_g, fc1_b, fc2_W, fc2_g, fc2_b, lin_W, lin_g, lin_b):
    raise NotImplementedError("write your pallas kernel here")


# speedup vs baseline: 1.0175x; 1.0175x over previous
import functools

import jax
import jax.numpy as jnp
from jax.experimental import pallas as pl
from jax.experimental.pallas import tpu as pltpu

N_NODES = 10000
N_MAP = 128
ROW_BLK = 2000
_EPS = 1e-5


def _gn(x, g, b):
    mu = jnp.mean(x, axis=1, keepdims=True)
    var = jnp.mean((x - mu) ** 2, axis=1, keepdims=True)
    return (x - mu) * jax.lax.rsqrt(var + _EPS) * g + b


def _in_mlp_body(x_ref, w1_ref, g1_ref, b1_ref, w2_ref, g2_ref, b2_ref,
                 wt_ref, gt_ref, bt_ref, o_ref):
    x = x_ref[...]
    h = jax.nn.relu(_gn(jnp.dot(x, w1_ref[...],
                                preferred_element_type=jnp.float32),
                        g1_ref[...], b1_ref[...]))
    h2 = _gn(jnp.dot(h, w2_ref[...], preferred_element_type=jnp.float32),
             g2_ref[...], b2_ref[...])
    t = _gn(jnp.dot(x, wt_ref[...], preferred_element_type=jnp.float32),
            gt_ref[...], bt_ref[...])
    o_ref[...] = jax.nn.relu(h2 + t)


def _input_mlp(feats, w1, g1, b1, w2, g2, b2, wt, gt, bt):
    n = feats.shape[0]
    grid = n // ROW_BLK
    row_spec = pl.BlockSpec((ROW_BLK, feats.shape[1]), lambda i: (i, 0))
    full = lambda a: pl.BlockSpec(a.shape, lambda i: (0,) * a.ndim)
    return pl.pallas_call(
        _in_mlp_body,
        grid=(grid,),
        in_specs=[row_spec] + [full(a) for a in (w1, g1, b1, w2, g2, b2, wt, gt, bt)],
        out_specs=pl.BlockSpec((ROW_BLK, N_MAP), lambda i: (i, 0)),
        out_shape=jax.ShapeDtypeStruct((n, N_MAP), jnp.float32),
    )(feats, w1, g1, b1, w2, g2, b2, wt, gt, bt)


def _pre_body(x_ref, w_ref, g_ref, b_ref, o_ref):
    o_ref[...] = jax.nn.relu(
        _gn(jnp.dot(x_ref[...], w_ref[...], preferred_element_type=jnp.float32),
            g_ref[...], b_ref[...]))


def _pre(feat, w, g, b):
    n = feat.shape[0]
    full = lambda a: pl.BlockSpec(a.shape, lambda i: (0,) * a.ndim)
    return pl.pallas_call(
        _pre_body,
        grid=(n // ROW_BLK,),
        in_specs=[pl.BlockSpec((ROW_BLK, N_MAP), lambda i: (i, 0)),
                  full(w), full(g), full(b)],
        out_specs=pl.BlockSpec((ROW_BLK, N_MAP), lambda i: (i, 0)),
        out_shape=jax.ShapeDtypeStruct((n, N_MAP), jnp.float32),
    )(feat, w, g, b)


def _post_body(feat_ref, agg_ref, wa_ref, wb_ref, g2_ref, b2_ref,
               wl_ref, gl_ref, bl_ref, o_ref):
    feat = feat_ref[...]
    x = (jnp.dot(feat, wa_ref[...], preferred_element_type=jnp.float32)
         + jnp.dot(agg_ref[...], wb_ref[...], preferred_element_type=jnp.float32))
    x = jax.nn.relu(_gn(x, g2_ref[...], b2_ref[...]))
    x = _gn(jnp.dot(x, wl_ref[...], preferred_element_type=jnp.float32),
            gl_ref[...], bl_ref[...])
    o_ref[...] = jax.nn.relu(x + feat)


def _post(feat, agg, w2, g2, b2, wl, gl, bl):
    n = feat.shape[0]
    wa, wb = w2[:N_MAP], w2[N_MAP:]
    full = lambda a: pl.BlockSpec(a.shape, lambda i: (0,) * a.ndim)
    row = pl.BlockSpec((ROW_BLK, N_MAP), lambda i: (i, 0))
    return pl.pallas_call(
        _post_body,
        grid=(n // ROW_BLK,),
        in_specs=[row, row, full(wa), full(wb), full(g2), full(b2),
                  full(wl), full(gl), full(bl)],
        out_specs=row,
        out_shape=jax.ShapeDtypeStruct((n, N_MAP), jnp.float32),
    )(feat, agg, wa, wb, g2, b2, wl, gl, bl)


def _seg_max(ctx, u, v):
    gathered = jnp.take(ctx, u, axis=0)
    agg = jax.ops.segment_max(gathered, v, num_segments=N_NODES)
    return jnp.where(agg == -jnp.inf, 0.0, agg)


def kernel(feats, u, v, in_W1, in_g1, in_b1, in_W2, in_g2, in_b2, in_Wt,
           in_gt, in_bt, fc1_W, fc1_g, fc1_b, fc2_W, fc2_g, fc2_b,
           lin_W, lin_g, lin_b):
    feat = _input_mlp(feats, in_W1, in_g1, in_b1, in_W2, in_g2, in_b2,
                      in_Wt, in_gt, in_bt)
    n_scales = u.shape[0]
    n_blk = fc1_W.shape[0] // n_scales
    for j in range(n_blk):
        for i in range(n_scales):
            t = j * n_scales + i
            ctx = _pre(feat, fc1_W[t], fc1_g[t], fc1_b[t])
            agg = _seg_max(ctx, u[i], v[i])
            feat = _post(feat, agg, fc2_W[t], fc2_g[t], fc2_b[t],
                         lin_W[t], lin_g[t], lin_b[t])
    return feat
